# baseline (speedup 1.0000x reference)
import functools

import jax
import jax.numpy as jnp
from jax import lax
from jax.experimental import pallas as pl
from jax.experimental.pallas import tpu as pltpu
from jax.experimental.pallas import tpu_sc as plsc

VOCAB = 1000000
DIM = 64
B = 16384
L = 20

NC = 2
NS = 16
NW = NC * NS

BW = B // NW
RW = BW * L
NCB = B // 128
CBW = NCB // NW
NIT = CBW * L
NSLOT = 2
NRB = DIM // 8


def _sc_kernel(eids_hbm, rids_hbm, fids_hbm, etab_hbm, rtab_hbm, ftab_hbm,
               eout_hbm, rout_hbm, fout_hbm,
               eidx_v, ridx_v, fidx_v, gbuf, tbe, tbr, fbT, iidx,
               rtab_v, ftab_v, colbuf_v,
               gsem0, gsem1, osem0, osem1, fsem):
    sid = lax.axis_index("s")
    wid = sid * NC + lax.axis_index("c")
    rbase = wid * RW
    bbase = wid * BW
    cbg0 = wid * CBW
    gsem = (gsem0, gsem1)
    osem = (osem0, osem1)

    pltpu.sync_copy(rtab_hbm, rtab_v)
    pltpu.sync_copy(ftab_hbm, ftab_v)

    pltpu.sync_copy(eids_hbm.at[pl.ds(rbase, RW)], eidx_v)
    pltpu.sync_copy(rids_hbm.at[pl.ds(rbase, RW)], ridx_v)
    pltpu.sync_copy(fids_hbm.at[pl.ds(bbase, BW)], fidx_v)

    iota16 = lax.iota(jnp.int32, 16)
    b16s = [iota16 + jb * 16 for jb in range(8)]

    def colinit(d, carry):
        colbuf_v[pl.ds(d * 16, 16)] = jnp.zeros((16,), jnp.int32) + d
        return carry

    lax.fori_loop(0, DIM, colinit, 0)

    def frame_blk(cbl, carry):
        fid16s = [fidx_v[pl.ds(cbl * 128 + jb * 16, 16)] for jb in range(8)]

        def fcol(rb, carry2):
            frow = fbT.at[rb]
            cbase = colbuf_v.at[pl.ds(rb * 128, 128)]
            for r in range(8):
                colv = cbase[pl.ds(r * 16, 16)]
                for jb in range(8):
                    v = plsc.load_gather(ftab_v, [fid16s[jb], colv])
                    frow[r, pl.ds(jb * 16, 16)] = v
            return carry2

        lax.fori_loop(0, NRB, fcol, 0)
        pltpu.async_copy(fbT, fout_hbm.at[:, cbg0 + cbl], fsem).wait()
        return carry

    lax.fori_loop(0, CBW, frame_blk, 0)

    def build_idx(it, s):
        l = lax.rem(it, L)
        cbl = it // L
        for jb in range(8):
            pos = (cbl * 128 + jb * 16 + iota16) * L + l
            iidx.at[s].at[0][pl.ds(jb * 16, 16)] = \
                plsc.load_gather(eidx_v, [pos])
            iidx.at[s].at[1][pl.ds(jb * 16, 16)] = \
                plsc.load_gather(ridx_v, [pos])

    def gather_desc(s):
        return pltpu.make_async_copy(etab_hbm.at[iidx.at[s].at[0]],
                                     gbuf.at[s], gsem[s])

    def out_descs(it, s):
        l = lax.rem(it, L)
        cbg = cbg0 + it // L
        return [
            pltpu.make_async_copy(tbe.at[s], eout_hbm.at[l].at[:, cbg],
                                  osem[s]),
            pltpu.make_async_copy(tbr.at[s], rout_hbm.at[l].at[:, cbg],
                                  osem[s]),
        ]

    def build_item(it, s):
        cbl = it // L
        fid16s = [fidx_v[pl.ds(cbl * 128 + jb * 16, 16)] for jb in range(8)]
        rid16s = [iidx.at[s].at[1][pl.ds(jb * 16, 16)] for jb in range(8)]
        gblk = gbuf.at[s]
        eblk = tbe.at[s]
        rblk = tbr.at[s]
        def col(rb, carry2):
            erow = eblk.at[rb]
            rrow = rblk.at[rb]
            cbase = colbuf_v.at[pl.ds(rb * 128, 128)]
            for r in range(8):
                colv = cbase[pl.ds(r * 16, 16)]
                for jb in range(8):
                    ev = plsc.load_gather(gblk, [b16s[jb], colv])
                    fv = plsc.load_gather(ftab_v, [fid16s[jb], colv])
                    erow[r, pl.ds(jb * 16, 16)] = ev + fv
                    rv = plsc.load_gather(rtab_v, [rid16s[jb], colv])
                    rrow[r, pl.ds(jb * 16, 16)] = rv
            return carry2

        lax.fori_loop(0, NRB, col, 0)

    build_idx(0, 0)
    gather_desc(0).start()

    def pair_body(p, carry):
        for s in (0, 1):
            it = p * 2 + s
            s2 = 1 - s

            @pl.when(it >= 1)
            def _():
                for d in out_descs(it - 1, s2):
                    d.wait()

            @pl.when(it + 1 < NIT)
            def _():
                build_idx(it + 1, s2)
                gather_desc(s2).start()

            gather_desc(s).wait()
            build_item(it, s)
            for d in out_descs(it, s):
                d.start()
        return carry

    lax.fori_loop(0, NIT // 2, pair_body, 0)
    for d in out_descs(NIT - 1, 1):
        d.wait()


@jax.jit
def _encode(element_ids, relation_ids, frame_id, element_embed,
            relation_embed, frame_embed):
    mesh = plsc.VectorSubcoreMesh(core_axis_name="c", subcore_axis_name="s",
                                  num_cores=NC, num_subcores=NS)
    f32 = jnp.float32
    run = functools.partial(
        pl.kernel,
        out_type=(
            jax.ShapeDtypeStruct((L, NRB, NCB, 8, 128), f32),
            jax.ShapeDtypeStruct((L, NRB, NCB, 8, 128), f32),
            jax.ShapeDtypeStruct((NRB, NCB, 8, 128), f32),
        ),
        mesh=mesh,
        compiler_params=pltpu.CompilerParams(use_tc_tiling_on_sc=False,
                                             needs_layout_passes=False),
        scratch_types=[
            pltpu.VMEM((RW,), jnp.int32),
            pltpu.VMEM((RW,), jnp.int32),
            pltpu.VMEM((BW,), jnp.int32),
            pltpu.VMEM((NSLOT, 128, DIM), f32),
            pltpu.VMEM((NSLOT, NRB, 8, 128), f32),
            pltpu.VMEM((NSLOT, NRB, 8, 128), f32),
            pltpu.VMEM((NRB, 8, 128), f32),
            pltpu.VMEM((NSLOT, 2, 128), jnp.int32),
            pltpu.VMEM((20, DIM), f32),
            pltpu.VMEM((100, DIM), f32),
            pltpu.VMEM((DIM * 16,), jnp.int32),
        ] + [pltpu.SemaphoreType.DMA] * 5,
    )(_sc_kernel)
    return run(element_ids.reshape(B * L), relation_ids.reshape(B * L),
               frame_id, element_embed, relation_embed, frame_embed)


def kernel(element_ids, relation_ids, frame_id, element_embed,
           relation_embed, frame_embed):
    e5, r5, f4 = _encode(element_ids, relation_ids, frame_id,
                         element_embed, relation_embed, frame_embed)
    elements = e5.transpose(2, 4, 0, 1, 3).reshape(B, L, DIM)
    relations = r5.transpose(2, 4, 0, 1, 3).reshape(B, L, DIM)
    frame = f4.transpose(1, 3, 0, 2).reshape(B, DIM)
    return (elements, relations, frame)


# speedup vs baseline: 1.1353x; 1.1353x over previous
import functools

import jax
import jax.numpy as jnp
from jax import lax
from jax.experimental import pallas as pl
from jax.experimental.pallas import tpu as pltpu
from jax.experimental.pallas import tpu_sc as plsc

VOCAB = 1000000
DIM = 64
B = 16384
L = 20

NC = 2
NS = 16
NW = NC * NS

BW = B // NW
CB = 16
NCHUNK = BW // CB
RW = BW * L
CR = CB * L
GSUB = ((0, 128), (128, 128), (256, 64))
NSLOT = 2


def _sc_kernel(eids_hbm, rids_hbm, fids_hbm, etab_hbm, rtab_hbm, ftab_hbm,
               eout_hbm, rout_hbm, fout_hbm,
               eidx_v, ridx_v, fidx_v, ebuf, rbuf, fbuf, rtab_v, ftab_v,
               gsem0, gsem1, gsem2, gsem3, osem0, osem1, osem2, osem3):
    sid = lax.axis_index("s")
    wid = sid * NC + lax.axis_index("c")
    rbase = wid * RW
    bbase = wid * BW
    gsem = (gsem0, gsem1, gsem2, gsem3)[:NSLOT]
    osem = (osem0, osem1, osem2, osem3)[:NSLOT]

    pltpu.sync_copy(rtab_hbm, rtab_v)
    pltpu.sync_copy(ftab_hbm, ftab_v)

    pltpu.sync_copy(eids_hbm.at[pl.ds(rbase, RW)], eidx_v)
    pltpu.sync_copy(rids_hbm.at[pl.ds(rbase, RW)], ridx_v)
    pltpu.sync_copy(fids_hbm.at[pl.ds(bbase, BW)], fidx_v.at[pl.ds(0, BW)])
    fidx_v[pl.ds(BW, 16)] = jnp.zeros((16,), jnp.int32)

    def gather_descs(c, s):
        crow = c * CR
        return [pltpu.make_async_copy(
            etab_hbm.at[eidx_v.at[pl.ds(crow + off, n)]],
            ebuf.at[s].at[pl.ds(off, n)], gsem[s]) for off, n in GSUB]

    iota16 = lax.iota(jnp.int32, 16)

    def build_small(c, s):
        cb0 = c * CB
        crow = c * CR
        fid16 = fidx_v[pl.ds(cb0, 16)]
        for d in range(DIM):
            colv = jnp.full((16,), d, jnp.int32)
            v = plsc.load_gather(ftab_v, [fid16, colv])
            plsc.store_scatter(fbuf.at[s], [iota16, colv], v)

        def grp(g, carry):
            rid16 = ridx_v[pl.ds(crow + g * 16, 16)]
            rowv = iota16 + g * 16
            for d in range(DIM):
                colv = jnp.full((16,), d, jnp.int32)
                v = plsc.load_gather(rtab_v, [rid16, colv])
                plsc.store_scatter(rbuf.at[s], [rowv, colv], v)
            return carry

        lax.fori_loop(0, CR // 16, grp, 0)

    def out_descs(c, s):
        crow = c * CR
        cb0 = c * CB
        return [
            pltpu.make_async_copy(fbuf.at[s].at[pl.ds(0, CB)],
                                  fout_hbm.at[pl.ds(bbase + cb0, CB)],
                                  osem[s]),
            pltpu.make_async_copy(ebuf.at[s],
                                  eout_hbm.at[pl.ds(rbase + crow, CR)],
                                  osem[s]),
            pltpu.make_async_copy(rbuf.at[s],
                                  rout_hbm.at[pl.ds(rbase + crow, CR)],
                                  osem[s]),
        ]

    def add_frame(s):
        def add_body(b, carry2):
            row0 = b * L
            for d in range(DIM // 16):
                fv = fbuf.at[s][b, pl.ds(d * 16, 16)]
                for l in range(L):
                    ebuf.at[s][row0 + l, pl.ds(d * 16, 16)] += fv
            return carry2

        lax.fori_loop(0, CB, add_body, 0)

    for k in range(NSLOT - 1):
        for d in gather_descs(k, k):
            d.start()

    def group_body(g, carry):
        for s in range(NSLOT):
            c = g * NSLOT + s

            @pl.when(c >= 1)
            def _():
                for d in out_descs(c - 1, (s - 1) % NSLOT):
                    d.wait()

            @pl.when(c + NSLOT - 1 < NCHUNK)
            def _():
                for d in gather_descs(c + NSLOT - 1, (s - 1) % NSLOT):
                    d.start()

            build_small(c, s)
            for d in gather_descs(c, s):
                d.wait()
            add_frame(s)
            for d in out_descs(c, s):
                d.start()
        return carry

    lax.fori_loop(0, NCHUNK // NSLOT, group_body, 0)
    for d in out_descs(NCHUNK - 1, (NCHUNK - 1) % NSLOT):
        d.wait()


@jax.jit
def _encode(element_ids, relation_ids, frame_id, element_embed,
            relation_embed, frame_embed):
    mesh = plsc.VectorSubcoreMesh(core_axis_name="c", subcore_axis_name="s",
                                  num_cores=NC, num_subcores=NS)
    f32 = jnp.float32
    run = functools.partial(
        pl.kernel,
        out_type=(
            jax.ShapeDtypeStruct((B * L, DIM), f32),
            jax.ShapeDtypeStruct((B * L, DIM), f32),
            jax.ShapeDtypeStruct((B, DIM), f32),
        ),
        mesh=mesh,
        compiler_params=pltpu.CompilerParams(use_tc_tiling_on_sc=False,
                                             needs_layout_passes=False),
        scratch_types=[
            pltpu.VMEM((RW,), jnp.int32),
            pltpu.VMEM((RW,), jnp.int32),
            pltpu.VMEM((BW + 16,), jnp.int32),
            pltpu.VMEM((NSLOT, CR, DIM), f32),
            pltpu.VMEM((NSLOT, CR, DIM), f32),
            pltpu.VMEM((NSLOT, 16, DIM), f32),
            pltpu.VMEM((20, DIM), f32),
            pltpu.VMEM((100, DIM), f32),
        ] + [pltpu.SemaphoreType.DMA] * 8,
    )(_sc_kernel)
    return run(element_ids.reshape(B * L), relation_ids.reshape(B * L),
               frame_id, element_embed, relation_embed, frame_embed)


def kernel(element_ids, relation_ids, frame_id, element_embed,
           relation_embed, frame_embed):
    eflat, rflat, frame = _encode(element_ids, relation_ids, frame_id,
                                  element_embed, relation_embed, frame_embed)
    return (eflat.reshape(B, L, DIM), rflat.reshape(B, L, DIM), frame)
